# baseline (speedup 1.0000x reference)
import functools

import jax
import jax.numpy as jnp
from jax.experimental import pallas as pl
from jax.experimental.pallas import tpu as pltpu

_B = 1


def _taps(w, cin_pad=None):
    cout, cin, _, _ = w.shape
    t = w.reshape(cout, cin, 9).transpose(2, 1, 0)
    if cin_pad is not None and cin_pad > cin:
        t = jnp.pad(t, ((0, 0), (0, cin_pad - cin), (0, 0)))
    return t


def _fused_convs_kernel(xp_ref, w1_ref, b1_ref, w2_ref, b2_ref, w3_ref, b3_ref,
                        o1_ref, o3_ref, pad2, pad3, s1, s2, s3):
    @pl.when(pl.program_id(0) == 0)
    def _zero_borders():
        pad2[...] = jnp.zeros_like(pad2)
        pad3[...] = jnp.zeros_like(pad3)

    def conv_pool(slab, wr, br, h, cout, s, d, do_relu):
        acc = None
        for t in range(9):
            kh, kw = divmod(t, 3)
            a = slab(kh, kw)
            dd = jnp.dot(a, wr[t], preferred_element_type=jnp.float32)
            acc = dd if acc is None else acc + dd
        y = acc + br[...]
        if do_relu:
            y = jnp.maximum(y, 0.0)
        hp = h // 2
        yh = y.reshape(hp, 2, h, cout)
        s[d] = jnp.maximum(yh[:, 0], yh[:, 1]).reshape(hp * h, cout)
        ev = s[d, pl.dslice(0, hp * hp, 2), :]
        od = s[d, pl.dslice(1, hp * hp, 2), :]
        return jnp.maximum(ev, od)

    for b in range(_B):
        d = b % 2
        p1 = conv_pool(
            lambda kh, kw: xp_ref[b, kh:kh + 96, kw:kw + 96, :].reshape(9216, 8),
            w1_ref, b1_ref, 96, 32, s1, d, False)
        o1_ref[b] = p1
        pad2[d, 1:49, 1:49, :] = p1.reshape(48, 48, 32)
        p2 = conv_pool(
            lambda kh, kw: pad2[d, kh:kh + 48, kw:kw + 48, :].reshape(2304, 32),
            w2_ref, b2_ref, 48, 64, s2, d, True)
        pad3[d, 1:25, 1:25, :] = p2.reshape(24, 24, 64)
        p3 = conv_pool(
            lambda kh, kw: pad3[d, kh:kh + 24, kw:kw + 24, :].reshape(576, 64),
            w3_ref, b3_ref, 24, 64, s3, d, True)
        o3_ref[b] = p3


def _head_kernel(a_ref, w1_ref, b1_ref, w2_ref, b2_ref, o_ref):
    h = jnp.dot(a_ref[...], w1_ref[...], preferred_element_type=jnp.float32)
    h = jnp.maximum(h + b1_ref[...], 0.0)
    o_ref[...] = (jnp.dot(h, w2_ref[...], preferred_element_type=jnp.float32)
                  + b2_ref[...])


def kernel(x, w1, b1, w2, b2, w3, b3, wd1, bd1, wd2, bd2):
    n = x.shape[0]
    xh = jnp.transpose(x, (0, 2, 3, 1))
    xp = jnp.pad(xh, ((0, 0), (1, 1), (1, 1), (0, 5)))

    w1t = _taps(w1, cin_pad=8)
    w2t = _taps(w2)
    w3t = _taps(w3)

    o1, o3 = pl.pallas_call(
        _fused_convs_kernel,
        out_shape=[
            jax.ShapeDtypeStruct((n, 2304, 32), jnp.float32),
            jax.ShapeDtypeStruct((n, 144, 64), jnp.float32),
        ],
        grid=(n // _B,),
        in_specs=[
            pl.BlockSpec((_B, 98, 98, 8), lambda i: (i, 0, 0, 0)),
            pl.BlockSpec((9, 8, 32), lambda i: (0, 0, 0)),
            pl.BlockSpec((1, 32), lambda i: (0, 0)),
            pl.BlockSpec((9, 32, 64), lambda i: (0, 0, 0)),
            pl.BlockSpec((1, 64), lambda i: (0, 0)),
            pl.BlockSpec((9, 64, 64), lambda i: (0, 0, 0)),
            pl.BlockSpec((1, 64), lambda i: (0, 0)),
        ],
        out_specs=[
            pl.BlockSpec((_B, 2304, 32), lambda i: (i, 0, 0)),
            pl.BlockSpec((_B, 144, 64), lambda i: (i, 0, 0)),
        ],
        scratch_shapes=[
            pltpu.VMEM((2, 50, 50, 32), jnp.float32),
            pltpu.VMEM((2, 26, 26, 64), jnp.float32),
            pltpu.VMEM((2, 4608, 32), jnp.float32),
            pltpu.VMEM((2, 1152, 64), jnp.float32),
            pltpu.VMEM((2, 288, 64), jnp.float32),
        ],
        compiler_params=pltpu.CompilerParams(
            dimension_semantics=("arbitrary",)),
    )(xp, w1t, b1.reshape(1, 32), w2t, b2.reshape(1, 64),
      w3t, b3.reshape(1, 64))

    conv1_out = jnp.transpose(o1.reshape(n, 48, 48, 32), (0, 3, 1, 2))

    a = jnp.transpose(o3, (0, 2, 1)).reshape(n, 9216)
    logits = pl.pallas_call(
        _head_kernel,
        out_shape=jax.ShapeDtypeStruct((n, 4), jnp.float32),
        grid=(2,),
        in_specs=[
            pl.BlockSpec((n // 2, 9216), lambda i: (i, 0)),
            pl.BlockSpec((9216, 128), lambda i: (0, 0)),
            pl.BlockSpec((1, 128), lambda i: (0, 0)),
            pl.BlockSpec((128, 4), lambda i: (0, 0)),
            pl.BlockSpec((1, 4), lambda i: (0, 0)),
        ],
        out_specs=pl.BlockSpec((n // 2, 4), lambda i: (i, 0)),
        compiler_params=pltpu.CompilerParams(
            dimension_semantics=("arbitrary",)),
    )(a, wd1.T, bd1.reshape(1, 128), wd2.T, bd2.reshape(1, 4))

    return logits, conv1_out


# speedup vs baseline: 3.4759x; 3.4759x over previous
import jax
import jax.numpy as jnp
from jax.experimental import pallas as pl
from jax.experimental.pallas import tpu as pltpu

_B = 1


def _taps(w, cin_pad=None):
    cout, cin, _, _ = w.shape
    t = w.reshape(cout, cin, 9).transpose(2, 1, 0)
    if cin_pad is not None and cin_pad > cin:
        t = jnp.pad(t, ((0, 0), (0, cin_pad - cin), (0, 0)))
    return t


def _fused_convs_kernel(x_ref, w1_ref, b1_ref, w2_ref, b2_ref, w3_ref, b3_ref,
                        o1_ref, o3_ref, pad1, pad2, pad3, s1, s2, s3):
    @pl.when(pl.program_id(0) == 0)
    def _zero_borders():
        pad1[...] = jnp.zeros_like(pad1)
        pad2[...] = jnp.zeros_like(pad2)
        pad3[...] = jnp.zeros_like(pad3)

    def conv_pool(slab, wr, br, h, cout, s, d, do_relu):
        acc = None
        for t in range(9):
            kh, kw = divmod(t, 3)
            a = slab(kh, kw)
            dd = jnp.dot(a, wr[t], preferred_element_type=jnp.float32)
            acc = dd if acc is None else acc + dd
        y = acc + br[...]
        if do_relu:
            y = jnp.maximum(y, 0.0)
        hp = h // 2
        yh = y.reshape(hp, 2, h, cout)
        s[d] = jnp.maximum(yh[:, 0], yh[:, 1]).reshape(hp * h, cout)
        ev = s[d, pl.dslice(0, hp * hp, 2), :]
        od = s[d, pl.dslice(1, hp * hp, 2), :]
        return jnp.maximum(ev, od)

    for b in range(_B):
        d = b % 2
        xt = jnp.transpose(x_ref[b])
        pad1[d, 1:97, 1:97, 0:3] = xt.reshape(96, 96, 3)
        p1 = conv_pool(
            lambda kh, kw: pad1[d, kh:kh + 96, kw:kw + 96, :].reshape(9216, 8),
            w1_ref, b1_ref, 96, 32, s1, d, False)
        o1_ref[b] = jnp.transpose(p1)
        pad2[d, 1:49, 1:49, :] = p1.reshape(48, 48, 32)
        p2 = conv_pool(
            lambda kh, kw: pad2[d, kh:kh + 48, kw:kw + 48, :].reshape(2304, 32),
            w2_ref, b2_ref, 48, 64, s2, d, True)
        pad3[d, 1:25, 1:25, :] = p2.reshape(24, 24, 64)
        p3 = conv_pool(
            lambda kh, kw: pad3[d, kh:kh + 24, kw:kw + 24, :].reshape(576, 64),
            w3_ref, b3_ref, 24, 64, s3, d, True)
        o3_ref[b] = jnp.transpose(p3)


def _head_kernel(a_ref, w1_ref, b1_ref, w2_ref, b2_ref, o_ref):
    h = jax.lax.dot_general(a_ref[...], w1_ref[...], (((1,), (1,)), ((), ())),
                            preferred_element_type=jnp.float32)
    h = jnp.maximum(h + b1_ref[...], 0.0)
    o_ref[...] = (jax.lax.dot_general(h, w2_ref[...], (((1,), (1,)), ((), ())),
                                      preferred_element_type=jnp.float32)
                  + b2_ref[...])


def kernel(x, w1, b1, w2, b2, w3, b3, wd1, bd1, wd2, bd2):
    n = x.shape[0]
    x3 = x.reshape(n, 3, 9216)

    w1t = _taps(w1, cin_pad=8)
    w2t = _taps(w2)
    w3t = _taps(w3)

    o1, o3 = pl.pallas_call(
        _fused_convs_kernel,
        out_shape=[
            jax.ShapeDtypeStruct((n, 32, 2304), jnp.float32),
            jax.ShapeDtypeStruct((n, 64, 144), jnp.float32),
        ],
        grid=(n // _B,),
        in_specs=[
            pl.BlockSpec((_B, 3, 9216), lambda i: (i, 0, 0)),
            pl.BlockSpec((9, 8, 32), lambda i: (0, 0, 0)),
            pl.BlockSpec((1, 32), lambda i: (0, 0)),
            pl.BlockSpec((9, 32, 64), lambda i: (0, 0, 0)),
            pl.BlockSpec((1, 64), lambda i: (0, 0)),
            pl.BlockSpec((9, 64, 64), lambda i: (0, 0, 0)),
            pl.BlockSpec((1, 64), lambda i: (0, 0)),
        ],
        out_specs=[
            pl.BlockSpec((_B, 32, 2304), lambda i: (i, 0, 0)),
            pl.BlockSpec((_B, 64, 144), lambda i: (i, 0, 0)),
        ],
        scratch_shapes=[
            pltpu.VMEM((2, 98, 98, 8), jnp.float32),
            pltpu.VMEM((2, 50, 50, 32), jnp.float32),
            pltpu.VMEM((2, 26, 26, 64), jnp.float32),
            pltpu.VMEM((2, 4608, 32), jnp.float32),
            pltpu.VMEM((2, 1152, 64), jnp.float32),
            pltpu.VMEM((2, 288, 64), jnp.float32),
        ],
        compiler_params=pltpu.CompilerParams(
            dimension_semantics=("arbitrary",)),
    )(x3, w1t, b1.reshape(1, 32), w2t, b2.reshape(1, 64),
      w3t, b3.reshape(1, 64))

    conv1_out = o1.reshape(n, 32, 48, 48)
    a = o3.reshape(n, 9216)

    logits = pl.pallas_call(
        _head_kernel,
        out_shape=jax.ShapeDtypeStruct((n, 4), jnp.float32),
        grid=(2,),
        in_specs=[
            pl.BlockSpec((n // 2, 9216), lambda i: (i, 0)),
            pl.BlockSpec((128, 9216), lambda i: (0, 0)),
            pl.BlockSpec((1, 128), lambda i: (0, 0)),
            pl.BlockSpec((4, 128), lambda i: (0, 0)),
            pl.BlockSpec((1, 4), lambda i: (0, 0)),
        ],
        out_specs=pl.BlockSpec((n // 2, 4), lambda i: (i, 0)),
        compiler_params=pltpu.CompilerParams(
            dimension_semantics=("arbitrary",)),
    )(a, wd1, bd1.reshape(1, 128), wd2, bd2.reshape(1, 4))

    return logits, conv1_out
